# baseline (speedup 1.0000x reference)
import functools

import jax
import jax.numpy as jnp
from jax import lax
from jax.experimental import pallas as pl
from jax.experimental.pallas import tpu as pltpu
from jax.experimental.pallas import tpu_sc as plsc

NUM_CORES = 2
NUM_SUBCORES = 16
NW = NUM_CORES * NUM_SUBCORES


def _make_item_transform(ni, f, e, blk, interpret=False):
    def body(i_ref, w_ref, b_ref, o_ref):
        x = (
            jnp.dot(i_ref[...], w_ref[...], preferred_element_type=jnp.float32)
            + b_ref[...]
        ).astype(jnp.bfloat16)
        lo = lax.bitcast_convert_type(x[:, : e // 2], jnp.uint16)
        hi = lax.bitcast_convert_type(x[:, e // 2 :], jnp.uint16)
        w32 = lo.astype(jnp.uint32) | (hi.astype(jnp.uint32) << 16)
        o_ref[...] = lax.bitcast_convert_type(w32, jnp.int32)

    return pl.pallas_call(
        body,
        grid=(ni // blk,),
        in_specs=[
            pl.BlockSpec((blk, f), lambda i: (i, 0)),
            pl.BlockSpec((f, e), lambda i: (0, 0)),
            pl.BlockSpec((1, e), lambda i: (0, 0)),
        ],
        out_specs=pl.BlockSpec((blk, e // 2), lambda i: (i, 0)),
        out_shape=jax.ShapeDtypeStruct((ni, e // 2), jnp.int32),
        interpret=interpret,
    )


def _make_sc_gather(ni, nu, f, e, b, t, interpret=False):
    total = b * t
    rpw = total // NW
    ch = 128
    nch = rpw // ch
    upw = b // NW
    mesh = plsc.VectorSubcoreMesh(core_axis_name="c", subcore_axis_name="s", num_cores=NUM_CORES, num_subcores=NUM_SUBCORES)

    @functools.partial(
        pl.kernel,
        out_type=(
            jax.ShapeDtypeStruct((total, e // 2), jnp.int32),
            jax.ShapeDtypeStruct((b, f), jnp.float32),
        ),
        mesh=mesh,
        scratch_types=[
            pltpu.VMEM((nch, ch), jnp.int32),
            [pltpu.VMEM((ch, e // 2), jnp.int32) for _ in range(4)],
            pltpu.VMEM((upw,), jnp.int32),
            pltpu.VMEM((upw, f), jnp.float32),
            [pltpu.SemaphoreType.DMA for _ in range(4)],
            [pltpu.SemaphoreType.DMA for _ in range(4)],
            pltpu.SemaphoreType.DMA,
        ],
        compiler_params=pltpu.CompilerParams(use_tc_tiling_on_sc=False),
        interpret=interpret,
    )
    def gather(iw_hbm, uw_hbm, nidx_hbm, nodes_hbm, nout_hbm, uout_hbm,
               idx_v, bufs, uidx_v, urows_v, gsems, ssems, usem):
        nbuf = 4
        wid = lax.axis_index("s") * NUM_CORES + lax.axis_index("c")
        base = wid * rpw
        pltpu.sync_copy(nidx_hbm.at[pl.ds(wid * nch, nch)], idx_v)
        ubase = wid * upw
        pltpu.sync_copy(nodes_hbm.at[pl.ds(ubase, upw)], uidx_v)
        ucopy = pltpu.async_copy(uw_hbm.at[uidx_v], urows_v, usem)
        g = [None] * nch
        s_ = [None] * nch

        def gat(k):
            return pltpu.async_copy(iw_hbm.at[idx_v.at[k]], bufs[k % nbuf],
                                    gsems[k % nbuf])

        def sto(k):
            return pltpu.async_copy(bufs[k % nbuf],
                                    nout_hbm.at[pl.ds(base + k * ch, ch)],
                                    ssems[k % nbuf])

        issued = 0
        for k in range(nch):
            while issued < min(k + nbuf, nch):
                if issued >= nbuf:
                    s_[issued - nbuf].wait()
                g[issued] = gat(issued)
                issued += 1
            g[k].wait()
            s_[k] = sto(k)
        for k in range(nch - nbuf, nch):
            s_[k].wait()
        ucopy.wait()
        pltpu.sync_copy(urows_v, uout_hbm.at[pl.ds(ubase, upw)])

    return gather


def _make_winner(b, blk, interpret=False):
    def body(seg_ref, all_ref, o_ref):
        a = seg_ref[...]
        row = all_ref[...]
        j = lax.broadcasted_iota(jnp.int32, (blk, b), 1)
        cand = jnp.where(a == row, j, -1)
        o_ref[...] = jnp.max(cand, axis=1, keepdims=True)

    return pl.pallas_call(
        body,
        grid=(b // blk,),
        in_specs=[
            pl.BlockSpec((blk, 1), lambda i: (i, 0)),
            pl.BlockSpec((1, b), lambda i: (0, 0)),
        ],
        out_specs=pl.BlockSpec((blk, 1), lambda i: (i, 0)),
        out_shape=jax.ShapeDtypeStruct((b, 1), jnp.int32),
        interpret=interpret,
    )


def _make_attention(b, t, f, e, nb, interpret=False):
    def body(u_ref, n_ref, wu_ref, bu_ref, atop_ref, abot_ref, b1_ref,
             w2_ref, nf_ref, emb_ref):
        nf = (
            jnp.dot(u_ref[...], wu_ref[...], preferred_element_type=jnp.float32)
            + bu_ref[...]
        )
        nf_ref[...] = nf
        c = (
            jnp.dot(nf, abot_ref[...], preferred_element_type=jnp.float32)
            + b1_ref[...]
        )
        wu = lax.bitcast_convert_type(n_ref[...], jnp.uint32)
        lo = lax.bitcast_convert_type((wu & 0xFFFF).astype(jnp.uint16),
                                      jnp.bfloat16)
        hi = lax.bitcast_convert_type((wu >> 16).astype(jnp.uint16),
                                      jnp.bfloat16)
        nb2 = jnp.concatenate([lo, hi], axis=1)
        pre = jnp.dot(nb2, atop_ref[...].astype(jnp.bfloat16),
                      preferred_element_type=jnp.float32)
        crep = jnp.broadcast_to(c[:, None, :], (nb, t, e)).reshape(nb * t, e)
        h = jnp.maximum(pre + crep, 0.0)
        s = jnp.dot(h, w2_ref[...], preferred_element_type=jnp.float32)
        s3 = s.reshape(nb, t, 1)
        m = jnp.max(s3, axis=1, keepdims=True)
        p3 = jnp.exp(s3 - m)
        den = jnp.sum(p3, axis=1, keepdims=True)
        wsum = jnp.sum(p3 * nb2.reshape(nb, t, e).astype(jnp.float32), axis=1)
        emb_ref[...] = wsum / den.reshape(nb, 1)

    return pl.pallas_call(
        body,
        grid=(b // nb,),
        in_specs=[
            pl.BlockSpec((nb, f), lambda i: (i, 0)),
            pl.BlockSpec((nb * t, e // 2), lambda i: (i, 0)),
            pl.BlockSpec((f, e), lambda i: (0, 0)),
            pl.BlockSpec((1, e), lambda i: (0, 0)),
            pl.BlockSpec((e, e), lambda i: (0, 0)),
            pl.BlockSpec((e, e), lambda i: (0, 0)),
            pl.BlockSpec((1, e), lambda i: (0, 0)),
            pl.BlockSpec((e, 1), lambda i: (0, 0)),
        ],
        out_specs=[
            pl.BlockSpec((nb, e), lambda i: (i, 0)),
            pl.BlockSpec((nb, e), lambda i: (i, 0)),
        ],
        out_shape=[
            jax.ShapeDtypeStruct((b, e), jnp.float32),
            jax.ShapeDtypeStruct((b, e), jnp.float32),
        ],
        interpret=interpret,
    )


def _make_sc_scatter(b, e, nu, interpret=False):
    upw = b // NW
    mesh = plsc.VectorSubcoreMesh(core_axis_name="c", subcore_axis_name="s", num_cores=NUM_CORES, num_subcores=NUM_SUBCORES)

    @functools.partial(
        pl.kernel,
        out_type=(),
        mesh=mesh,
        scratch_types=[
            pltpu.VMEM((upw,), jnp.int32),
            pltpu.VMEM((upw,), jnp.int32),
            pltpu.VMEM((upw, e), jnp.float32),
            pltpu.SemaphoreType.DMA,
        ],
        interpret=interpret,
    )
    def scatter(emb_hbm, w_hbm, nodes_hbm, out_hbm, widx_v, nidx_v, vals_v,
                sem):
        wid = lax.axis_index("s") * NUM_CORES + lax.axis_index("c")
        base = wid * upw
        pltpu.sync_copy(w_hbm.at[pl.ds(base, upw)], widx_v)
        pltpu.async_copy(emb_hbm.at[widx_v], vals_v, sem).wait()
        pltpu.sync_copy(nodes_hbm.at[pl.ds(base, upw)], nidx_v)
        pltpu.async_copy(vals_v, out_hbm.at[nidx_v], sem).wait()

    return scatter


def kernel(u_weight, i_weight, W_u, b_u, W_i, b_i, att_w1, att_b1, att_w2,
           att_b2, nodes, neigh_indices):
    nu, f = u_weight.shape
    ni = i_weight.shape[0]
    e = W_u.shape[1]
    b, t = neigh_indices.shape

    iW = _make_item_transform(ni, f, e, blk=2000)(i_weight, W_i,
                                                  b_i.reshape(1, e))

    nidx2 = neigh_indices.reshape(-1, 128)
    neighs_flat, u_rows = _make_sc_gather(ni, nu, f, e, b, t)(
        iW, u_weight, nidx2, nodes)

    w_idx = _make_winner(b, blk=256)(nodes.reshape(b, 1), nodes.reshape(1, b))
    w_idx = w_idx.reshape(b)

    nodes_fea, emb = _make_attention(b, t, f, e, nb=256)(
        u_rows, neighs_flat, W_u, b_u.reshape(1, e), att_w1[:e], att_w1[e:],
        att_b1.reshape(1, e), att_w2)

    out_ref = jax.new_ref(jnp.zeros((nu, e), jnp.float32))
    _make_sc_scatter(b, e, nu)(emb, w_idx, nodes, out_ref)
    return nodes_fea, out_ref[...]


# speedup vs baseline: 1.6872x; 1.6872x over previous
import functools

import jax
import jax.numpy as jnp
from jax import lax
from jax.experimental import pallas as pl
from jax.experimental.pallas import tpu as pltpu
from jax.experimental.pallas import tpu_sc as plsc

NUM_CORES = 2
NUM_SUBCORES = 16
NW = NUM_CORES * NUM_SUBCORES
_MESH = dict(core_axis_name="c", subcore_axis_name="s",
             num_cores=NUM_CORES, num_subcores=NUM_SUBCORES)


def _make_sc_gather(ni, nu, f, b, t, interpret=False):
    total = b * t
    rpw = total // NW
    ch = 128
    nch = rpw // ch
    nbuf = 3
    upw = b // NW
    uch = upw // 2
    mesh = plsc.VectorSubcoreMesh(**_MESH)

    @functools.partial(
        pl.kernel,
        out_type=(
            jax.ShapeDtypeStruct((total, f), jnp.float32),
            jax.ShapeDtypeStruct((b, f), jnp.float32),
        ),
        mesh=mesh,
        scratch_types=[
            pltpu.VMEM((nch, ch), jnp.int32),
            [pltpu.VMEM((ch, f), jnp.float32) for _ in range(3)],
            pltpu.VMEM((upw,), jnp.int32),
            pltpu.VMEM((upw // 2, f), jnp.float32),
            [pltpu.SemaphoreType.DMA for _ in range(3)],
            [pltpu.SemaphoreType.DMA for _ in range(3)],
            pltpu.SemaphoreType.DMA,
        ],
        interpret=interpret,
    )
    def gather(iw_hbm, uw_hbm, nidx_hbm, nodes_hbm, nout_hbm, uout_hbm,
               idx_v, bufs, uidx_v, urows_v, gsems, ssems, usem):
        wid = lax.axis_index("s") * NUM_CORES + lax.axis_index("c")
        base = wid * rpw
        pltpu.sync_copy(nidx_hbm.at[pl.ds(wid * nch, nch)], idx_v)
        ubase = wid * upw
        pltpu.sync_copy(nodes_hbm.at[pl.ds(ubase, upw)], uidx_v)
        g = [None] * nch
        s_ = [None] * nch

        def gat(k):
            return pltpu.async_copy(iw_hbm.at[idx_v.at[k]], bufs[k % nbuf],
                                    gsems[k % nbuf])

        def sto(k):
            return pltpu.async_copy(bufs[k % nbuf],
                                    nout_hbm.at[pl.ds(base + k * ch, ch)],
                                    ssems[k % nbuf])

        issued = 0
        for k in range(nch):
            while issued < min(k + nbuf, nch):
                if issued >= nbuf:
                    s_[issued - nbuf].wait()
                g[issued] = gat(issued)
                issued += 1
            g[k].wait()
            s_[k] = sto(k)
        for k in range(nch - nbuf, nch):
            s_[k].wait()
        for half in range(2):
            pltpu.async_copy(
                uw_hbm.at[uidx_v.at[pl.ds(half * uch, uch)]], urows_v,
                usem).wait()
            pltpu.sync_copy(urows_v,
                            uout_hbm.at[pl.ds(ubase + half * uch, uch)])

    return gather


def _make_winner(b, blk, interpret=False):
    def body(seg_ref, all_ref, o_ref):
        a = seg_ref[...]
        row = all_ref[...]
        j = lax.broadcasted_iota(jnp.int32, (blk, b), 1)
        cand = jnp.where(a == row, j, -1)
        o_ref[...] = jnp.max(cand, axis=1, keepdims=True)

    return pl.pallas_call(
        body,
        grid=(b // blk,),
        in_specs=[
            pl.BlockSpec((blk, 1), lambda i: (i, 0)),
            pl.BlockSpec((1, b), lambda i: (0, 0)),
        ],
        out_specs=pl.BlockSpec((blk, 1), lambda i: (i, 0)),
        out_shape=jax.ShapeDtypeStruct((b, 1), jnp.int32),
        interpret=interpret,
    )


def _make_attention(b, t, f, e, nb, interpret=False):
    def body(u_ref, n_ref, wu_ref, bu_ref, wi_ref, bi_ref, atop_ref,
             abot_ref, b1_ref, w2_ref, nf_ref, emb_ref):
        nf = (
            jnp.dot(u_ref[...], wu_ref[...], preferred_element_type=jnp.float32)
            + bu_ref[...]
        )
        nf_ref[...] = nf
        c = (
            jnp.dot(nf, abot_ref[...], preferred_element_type=jnp.float32)
            + b1_ref[...]
        )
        nb2 = (
            jnp.dot(n_ref[...].astype(jnp.bfloat16),
                    wi_ref[...].astype(jnp.bfloat16),
                    preferred_element_type=jnp.float32)
            + bi_ref[...]
        )
        pre = jnp.dot(nb2, atop_ref[...], preferred_element_type=jnp.float32)
        crep = jnp.broadcast_to(c[:, None, :], (nb, t, e)).reshape(nb * t, e)
        h = jnp.maximum(pre + crep, 0.0)
        s = jnp.dot(h, w2_ref[...], preferred_element_type=jnp.float32)
        s3 = s.reshape(nb, t, 1)
        m = jnp.max(s3, axis=1, keepdims=True)
        p3 = jnp.exp(s3 - m)
        den = jnp.sum(p3, axis=1, keepdims=True)
        wsum = jnp.sum(p3 * nb2.reshape(nb, t, e), axis=1)
        emb_ref[...] = wsum / den.reshape(nb, 1)

    return pl.pallas_call(
        body,
        grid=(b // nb,),
        in_specs=[
            pl.BlockSpec((nb, f), lambda i: (i, 0)),
            pl.BlockSpec((nb * t, f), lambda i: (i, 0)),
            pl.BlockSpec((f, e), lambda i: (0, 0)),
            pl.BlockSpec((1, e), lambda i: (0, 0)),
            pl.BlockSpec((f, e), lambda i: (0, 0)),
            pl.BlockSpec((1, e), lambda i: (0, 0)),
            pl.BlockSpec((e, e), lambda i: (0, 0)),
            pl.BlockSpec((e, e), lambda i: (0, 0)),
            pl.BlockSpec((1, e), lambda i: (0, 0)),
            pl.BlockSpec((e, 1), lambda i: (0, 0)),
        ],
        out_specs=[
            pl.BlockSpec((nb, e), lambda i: (i, 0)),
            pl.BlockSpec((nb, e), lambda i: (i, 0)),
        ],
        out_shape=[
            jax.ShapeDtypeStruct((b, e), jnp.float32),
            jax.ShapeDtypeStruct((b, e), jnp.float32),
        ],
        interpret=interpret,
    )


def _make_sc_scatter(b, e, nu, interpret=False):
    upw = b // NW
    mesh = plsc.VectorSubcoreMesh(**_MESH)

    @functools.partial(
        pl.kernel,
        out_type=(),
        mesh=mesh,
        scratch_types=[
            pltpu.VMEM((upw,), jnp.int32),
            pltpu.VMEM((upw,), jnp.int32),
            pltpu.VMEM((upw, e), jnp.float32),
            pltpu.SemaphoreType.DMA,
        ],
        interpret=interpret,
    )
    def scatter(emb_hbm, w_hbm, nodes_hbm, out_hbm, widx_v, nidx_v, vals_v,
                sem):
        wid = lax.axis_index("s") * NUM_CORES + lax.axis_index("c")
        base = wid * upw
        pltpu.sync_copy(w_hbm.at[pl.ds(base, upw)], widx_v)
        pltpu.async_copy(emb_hbm.at[widx_v], vals_v, sem).wait()
        pltpu.sync_copy(nodes_hbm.at[pl.ds(base, upw)], nidx_v)
        pltpu.async_copy(vals_v, out_hbm.at[nidx_v], sem).wait()

    return scatter


def kernel(u_weight, i_weight, W_u, b_u, W_i, b_i, att_w1, att_b1, att_w2,
           att_b2, nodes, neigh_indices):
    nu, f = u_weight.shape
    ni = i_weight.shape[0]
    e = W_u.shape[1]
    b, t = neigh_indices.shape

    nidx2 = neigh_indices.reshape(-1, 128)
    neighs_raw, u_rows = _make_sc_gather(ni, nu, f, b, t)(
        i_weight, u_weight, nidx2, nodes)

    w_idx = _make_winner(b, blk=256)(nodes.reshape(b, 1), nodes.reshape(1, b))
    w_idx = w_idx.reshape(b)

    nodes_fea, emb = _make_attention(b, t, f, e, nb=256)(
        u_rows, neighs_raw, W_u, b_u.reshape(1, e), W_i, b_i.reshape(1, e),
        att_w1[:e], att_w1[e:], att_b1.reshape(1, e), att_w2)

    out_ref = jax.new_ref(jnp.zeros((nu, e), jnp.float32))
    _make_sc_scatter(b, e, nu)(emb, w_idx, nodes, out_ref)
    return nodes_fea, out_ref[...]
